# baseline (speedup 1.0000x reference)
import functools

import jax
import jax.numpy as jnp
from jax import lax
from jax.experimental import pallas as pl
from jax.experimental.pallas import tpu as pltpu
from jax.experimental.pallas import tpu_sc as plsc


def _embed_kernel(B, n_cat, V, E):
    n_blk = E // 16
    n_tasks = n_cat * n_blk
    tasks_per_sc = n_tasks // 2
    qchunk = 2048
    hb = B // 2
    V128 = (V // 128) * 128
    vtail = V - V128
    plen = 8192
    pieces = [(i * plen, plen) for i in range(V128 // plen)]
    if V128 % plen:
        pieces.append(((V128 // plen) * plen, V128 % plen))
    n_pc = len(pieces)
    NBUF = 2
    NQ = 4
    qlen = plen // NQ
    mesh = plsc.VectorSubcoreMesh(core_axis_name="c", subcore_axis_name="s")

    @functools.partial(
        pl.kernel,
        out_type=jax.ShapeDtypeStruct((n_cat * E, B), jnp.float32),
        mesh=mesh,
        compiler_params=pltpu.CompilerParams(needs_layout_passes=False),
        scratch_types=[
            pltpu.VMEM_SHARED((16, plen), jnp.float32),
            pltpu.VMEM_SHARED((16, plen), jnp.float32),
            pltpu.VMEM_SHARED((16, hb), jnp.float32),
            pltpu.VMEM((V,), jnp.float32),
            pltpu.VMEM((16, 128), jnp.int32),
            pltpu.VMEM((qchunk,), jnp.float32),
            pltpu.VMEM((16, vtail), jnp.float32),
        ] + [pltpu.SemaphoreType.DMA] * (2 * 4),
    )
    def k(tab_hbm, cat_hbm, out_hbm, pbuf0, pbuf1, stage,
          row_v, cat_v, out_v, tail_v, *sems):
        c = lax.axis_index("c")
        t = lax.axis_index("s")
        task0 = c * tasks_per_sc
        bufs = (pbuf0, pbuf1)

        def piece_copy(j, g, i):
            poff, pln = pieces[i]
            b = i % NBUF
            if pln == plen:
                return [pltpu.make_async_copy(
                    tab_hbm.at[j, pl.ds(g * 16, 16),
                               pl.ds(poff + q * qlen, qlen)],
                    bufs[b].at[:, pl.ds(q * qlen, qlen)],
                    sems[b * NQ + q],
                ) for q in range(NQ)]
            return [pltpu.make_async_copy(
                tab_hbm.at[j, pl.ds(g * 16, 16), pl.ds(poff, pln)],
                bufs[b].at[:, pl.ds(0, pln)],
                sems[b * NQ],
            )]

        def issue(j, g, i):
            cps = piece_copy(j, g, i)
            for q in range(len(cps)):
                @pl.when(t == q)
                def _():
                    cps[q].start()

        def drain(j, g, i):
            cps = piece_copy(j, g, i)
            for q in range(len(cps)):
                @pl.when(t == q)
                def _():
                    cps[q].wait()

        @pl.loop(0, tasks_per_sc)
        def _task(p):
            tid = task0 + p
            j = tid // n_blk
            g = tid % n_blk

            issue(j, g, 0)
            for i, (poff, pln) in enumerate(pieces):
                drain(j, g, i)
                plsc.subcore_barrier()
                if i + 1 < n_pc:
                    issue(j, g, i + 1)
                b = i % NBUF
                if pln == plen:
                    pltpu.sync_copy(bufs[b].at[t, :],
                                    row_v.at[pl.ds(poff, pln)])
                else:
                    pltpu.sync_copy(bufs[b].at[t, pl.ds(0, pln)],
                                    row_v.at[pl.ds(poff, pln)])

            pltpu.sync_copy(
                tab_hbm.at[j, pl.ds(g * 16, 16), pl.ds(V128, vtail)], tail_v
            )
            for w in range(vtail // 16):
                row_v[pl.ds(V128 + w * 16, 16)] = tail_v[t, pl.ds(w * 16, 16)]
            plsc.subcore_barrier()

            for half in range(2):
                for qq in range(4):
                    pltpu.sync_copy(
                        cat_hbm.at[j, pl.ds(half * 64 + qq * 16, 16), :],
                        cat_v,
                    )

                    @pl.loop(0, 16)
                    def _rows(a):
                        for bb in range(8):
                            ii = cat_v[a, pl.ds(bb * 16, 16)]
                            vals = plsc.load_gather(row_v, [ii])
                            out_v[pl.ds(a * 128 + bb * 16, 16)] = vals

                    pltpu.sync_copy(
                        out_v, stage.at[t, pl.ds(qq * qchunk, qchunk)]
                    )
                plsc.subcore_barrier()

                @pl.when(t == 15)
                def _flush():
                    pltpu.sync_copy(
                        stage,
                        out_hbm.at[pl.ds(j * E + g * 16, 16),
                                   pl.ds(half * hb, hb)],
                    )
                plsc.subcore_barrier()

    return k


def kernel(x, tables):
    B, F = x.shape
    n_cat, V, E = tables.shape
    n_num = F - n_cat

    tab_t = jnp.transpose(tables, (0, 2, 1))
    cat_js = x[:, n_num:].astype(jnp.int32).T.reshape(n_cat, B // 128, 128)

    k = _embed_kernel(B, n_cat, V, E)
    emb_t = k(tab_t, cat_js)
    return jnp.concatenate([x[:, :n_num], emb_t.T], axis=1)


# speedup vs baseline: 1.1823x; 1.1823x over previous
import functools

import jax
import jax.numpy as jnp
from jax import lax
from jax.experimental import pallas as pl
from jax.experimental.pallas import tpu as pltpu
from jax.experimental.pallas import tpu_sc as plsc


def _embed_kernel(B, n_cat, V, E):
    n_blk = E // 16
    n_tasks = n_cat * n_blk
    tasks_per_sc = n_tasks // 2
    qchunk = 2048
    hb = B // 2
    V128 = (V // 128) * 128
    vtail = V - V128
    plen = 4096
    pieces = [(i * plen, plen) for i in range(V128 // plen)]
    if V128 % plen:
        pieces.append(((V128 // plen) * plen, V128 % plen))
    n_pc = len(pieces)
    NBUF = 4
    NQ = 2
    qlen = plen // NQ
    mesh = plsc.VectorSubcoreMesh(core_axis_name="c", subcore_axis_name="s")

    @functools.partial(
        pl.kernel,
        out_type=jax.ShapeDtypeStruct((n_cat * E, B), jnp.float32),
        mesh=mesh,
        compiler_params=pltpu.CompilerParams(needs_layout_passes=False),
        scratch_types=[
            pltpu.VMEM_SHARED((16, plen), jnp.float32),
            pltpu.VMEM_SHARED((16, plen), jnp.float32),
            pltpu.VMEM_SHARED((16, plen), jnp.float32),
            pltpu.VMEM_SHARED((16, plen), jnp.float32),
            pltpu.VMEM_SHARED((16, hb), jnp.float32),
            pltpu.VMEM((V,), jnp.float32),
            pltpu.VMEM((16, 128), jnp.int32),
            pltpu.VMEM((qchunk,), jnp.float32),
            pltpu.VMEM((16, vtail), jnp.float32),
        ] + [pltpu.SemaphoreType.DMA] * (4 * 2),
    )
    def k(tab_hbm, cat_hbm, out_hbm, pbuf0, pbuf1, pbuf2, pbuf3, stage,
          row_v, cat_v, out_v, tail_v, *sems):
        c = lax.axis_index("c")
        t = lax.axis_index("s")
        task0 = c * tasks_per_sc
        bufs = (pbuf0, pbuf1, pbuf2, pbuf3)

        def piece_copy(j, g, i):
            poff, pln = pieces[i]
            b = i % NBUF
            if pln == plen:
                return [pltpu.make_async_copy(
                    tab_hbm.at[j, pl.ds(g * 16, 16),
                               pl.ds(poff + q * qlen, qlen)],
                    bufs[b].at[:, pl.ds(q * qlen, qlen)],
                    sems[b * NQ + q],
                ) for q in range(NQ)]
            return [pltpu.make_async_copy(
                tab_hbm.at[j, pl.ds(g * 16, 16), pl.ds(poff, pln)],
                bufs[b].at[:, pl.ds(0, pln)],
                sems[b * NQ],
            )]

        def issue(j, g, i):
            cps = piece_copy(j, g, i)
            for q in range(len(cps)):
                @pl.when(t == q)
                def _():
                    cps[q].start()

        def drain(j, g, i):
            cps = piece_copy(j, g, i)
            for q in range(len(cps)):
                @pl.when(t == q)
                def _():
                    cps[q].wait()

        @pl.loop(0, tasks_per_sc)
        def _task(p):
            tid = task0 + p
            j = tid // n_blk
            g = tid % n_blk

            for i in range(NBUF - 1):
                issue(j, g, i)
            for i, (poff, pln) in enumerate(pieces):
                drain(j, g, i)
                plsc.subcore_barrier()
                if i + NBUF - 1 < n_pc:
                    issue(j, g, i + NBUF - 1)
                b = i % NBUF
                if pln == plen:
                    pltpu.sync_copy(bufs[b].at[t, :],
                                    row_v.at[pl.ds(poff, pln)])
                else:
                    pltpu.sync_copy(bufs[b].at[t, pl.ds(0, pln)],
                                    row_v.at[pl.ds(poff, pln)])

            pltpu.sync_copy(
                tab_hbm.at[j, pl.ds(g * 16, 16), pl.ds(V128, vtail)], tail_v
            )
            for w in range(vtail // 16):
                row_v[pl.ds(V128 + w * 16, 16)] = tail_v[t, pl.ds(w * 16, 16)]
            plsc.subcore_barrier()

            for half in range(2):
                for qq in range(4):
                    pltpu.sync_copy(
                        cat_hbm.at[j, pl.ds(half * 64 + qq * 16, 16), :],
                        cat_v,
                    )

                    @pl.loop(0, 16)
                    def _rows(a):
                        for bb in range(8):
                            ii = cat_v[a, pl.ds(bb * 16, 16)]
                            vals = plsc.load_gather(row_v, [ii])
                            out_v[pl.ds(a * 128 + bb * 16, 16)] = vals

                    pltpu.sync_copy(
                        out_v, stage.at[t, pl.ds(qq * qchunk, qchunk)]
                    )
                plsc.subcore_barrier()

                @pl.when(t == 15)
                def _flush():
                    pltpu.sync_copy(
                        stage,
                        out_hbm.at[pl.ds(j * E + g * 16, 16),
                                   pl.ds(half * hb, hb)],
                    )
                plsc.subcore_barrier()

    return k


def kernel(x, tables):
    B, F = x.shape
    n_cat, V, E = tables.shape
    n_num = F - n_cat

    tab_t = jnp.transpose(tables, (0, 2, 1))
    cat_js = x[:, n_num:].astype(jnp.int32).T.reshape(n_cat, B // 128, 128)

    k = _embed_kernel(B, n_cat, V, E)
    emb_t = k(tab_t, cat_js)
    return jnp.concatenate([x[:, :n_num], emb_t.T], axis=1)
